# baseline (speedup 1.0000x reference)
import functools

import jax
import jax.numpy as jnp
from jax import lax
from jax.experimental import pallas as pl
from jax.experimental.pallas import tpu as pltpu
from jax.experimental.pallas import tpu_sc as plsc

N = 10000
C = 20000
NP = 10240
CP = 20480
HP = CP // 2
BH = 3 * HP
NBLK = 1024
CBLK = 2048
NW = 32
RCH = 120
NCH = BH // NW // RCH
PW = NCH * RCH
SLAB = NP // 16
NBUF = 7
NBUF_S = 2


def _MM(x, w):
    bf = jnp.bfloat16
    xh = x.astype(bf)
    xl = (x - xh.astype(jnp.float32)).astype(bf)
    wh = w.astype(bf)
    wl = (w - wh.astype(jnp.float32)).astype(bf)
    d = functools.partial(jnp.dot, preferred_element_type=jnp.float32)
    return d(xh, wh) + (d(xh, wl) + d(xl, wh))


def _ln(h, scale, off):
    mu = jnp.mean(h, axis=-1, keepdims=True)
    d = h - mu
    var = jnp.mean(d * d, axis=-1, keepdims=True)
    return d * lax.rsqrt(var + 1e-5) * scale + off


def _sc_gather(table, idx3d, D):
    _, nch, rch = idx3d.shape
    pw = nch * rch
    mesh = plsc.VectorSubcoreMesh(core_axis_name="c", subcore_axis_name="s")

    def body(table_hbm, idx_hbm, out_hbm, idx_v, rows_v, gsem, osem):
        wid = lax.axis_index("s") * 2 + lax.axis_index("c")
        pltpu.sync_copy(idx_hbm.at[wid], idx_v)
        gd = [None] * NBUF
        od = [None] * NBUF
        lag = NBUF - 1
        for k in range(nch + lag):
            if k < nch:
                b = k % NBUF
                if od[b] is not None:
                    od[b].wait()
                gd[b] = pltpu.async_copy(table_hbm.at[idx_v.at[k]],
                                         rows_v.at[b], gsem.at[b])
            if lag <= k < nch + lag:
                kk = k - lag
                if kk < 0 or kk >= nch:
                    continue
                bb = kk % NBUF
                gd[bb].wait()
                od[bb] = pltpu.async_copy(
                    rows_v.at[bb],
                    out_hbm.at[pl.ds(wid * pw + kk * rch, rch)],
                    osem.at[bb])
        for b in range(NBUF):
            if od[b] is not None:
                od[b].wait()

    f = pl.kernel(
        body,
        out_type=jax.ShapeDtypeStruct((NW * pw, D), jnp.float32),
        mesh=mesh,
        scratch_types=[
            pltpu.VMEM((nch, rch), jnp.int32),
            pltpu.VMEM((NBUF, rch, D), jnp.float32),
            pltpu.SemaphoreType.DMA((NBUF,)),
            pltpu.SemaphoreType.DMA((NBUF,)),
        ],
    )
    return f(table, idx3d)


def _sc_scatter(vals, idx3d):
    mesh = plsc.VectorSubcoreMesh(core_axis_name="c", subcore_axis_name="s")

    def body(vals_hbm, idx_hbm, out_hbm, idx_v, vals_v, acc,
             isem, vsem, asem):
        cid = lax.axis_index("c")
        sid = lax.axis_index("s")
        wid = sid * 2 + cid

        idx_d = pltpu.async_copy(idx_hbm.at[wid], idx_v, isem)

        def zrow(i, carry):
            z = jnp.zeros((16,), jnp.float32)
            for jj in range(8):
                vals_v[0, i, pl.ds(jj * 16, 16)] = z
            return carry

        lax.fori_loop(0, RCH, zrow, 0)
        off = 0
        while off < SLAB:
            sz = min(RCH, SLAB - off)
            pltpu.sync_copy(vals_v.at[0, pl.ds(0, sz)],
                            acc.at[pl.ds(sid * SLAB + off, sz)])
            off += sz
        idx_d.wait()
        plsc.subcore_barrier()

        vd = [None] * NBUF_S
        ad = [None] * NBUF_S
        for k in range(NCH):
            b = k % NBUF_S
            if ad[b] is not None:
                ad[b].wait()
            vd[b] = pltpu.async_copy(
                vals_hbm.at[pl.ds(wid * PW + k * RCH, RCH)],
                vals_v.at[b], vsem.at[b])
            if k >= 1:
                bb = (k - 1) % NBUF_S
                vd[bb].wait()
                ad[bb] = pltpu.async_copy(vals_v.at[bb],
                                          acc.at[idx_v.at[k - 1]],
                                          asem.at[bb], add=True)
        bb = (NCH - 1) % NBUF_S
        vd[bb].wait()
        ad[bb] = pltpu.async_copy(vals_v.at[bb], acc.at[idx_v.at[NCH - 1]],
                                  asem.at[bb], add=True)
        for b in range(NBUF_S):
            if ad[b] is not None:
                ad[b].wait()
        plsc.subcore_barrier()
        pltpu.sync_copy(acc.at[pl.ds(sid * SLAB, SLAB)],
                        out_hbm.at[cid, pl.ds(sid * SLAB, SLAB)])

    f = pl.kernel(
        body,
        out_type=jax.ShapeDtypeStruct((2, NP, 128), jnp.float32),
        mesh=mesh,
        scratch_types=[
            pltpu.VMEM((NCH, RCH), jnp.int32),
            pltpu.VMEM((NBUF_S, RCH, 128), jnp.float32),
            pltpu.VMEM_SHARED((NP, 128), jnp.float32),
            pltpu.SemaphoreType.DMA,
            pltpu.SemaphoreType.DMA((NBUF_S,)),
            pltpu.SemaphoreType.DMA((NBUF_S,)),
        ],
    )
    return f(vals, idx3d)


def _f32(*shape):
    return jax.ShapeDtypeStruct(shape, jnp.float32)


def _wspec(*shp):
    nd = len(shp)
    return pl.BlockSpec(shp, lambda *_: (0,) * nd)


def _encode_body(wp, pwp, nt, wv, wo, b1, w2, b2, w3, b3, sc, of, o):
    vel = wp[...] - pwp[...]
    lane = lax.broadcasted_iota(jnp.int32, (NBLK, 16), 1)
    oh = (lane == nt[...]).astype(jnp.float32)
    h = _MM(vel, wv[...]) + _MM(oh, wo[...]) + b1[...]
    h = jax.nn.relu(h)
    h = jax.nn.relu(_MM(h, w2[...]) + b2[...])
    h = _MM(h, w3[...]) + b3[...]
    o[...] = _ln(h, sc[...], of[...])


def _encode(wp8, pwp8, nt, *w):
    return pl.pallas_call(
        _encode_body,
        grid=(NP // NBLK,),
        in_specs=[
            pl.BlockSpec((NBLK, 8), lambda nb: (nb, 0)),
            pl.BlockSpec((NBLK, 8), lambda nb: (nb, 0)),
            pl.BlockSpec((NBLK, 1), lambda nb: (nb, 0)),
            _wspec(8, 128), _wspec(16, 128), _wspec(1, 128),
            _wspec(128, 128), _wspec(1, 128), _wspec(128, 128),
            _wspec(1, 128), _wspec(1, 128), _wspec(1, 128),
        ],
        out_specs=pl.BlockSpec((NBLK, 128), lambda nb: (nb, 0)),
        out_shape=_f32(NP, 128),
    )(wp8, pwp8, nt, *w)


def _edge_enc_body(ps, pr, w16, wnw, wnm, b1, w2, b2, w3, b3, sc, of, o):
    rel = ps[0] - pr[0]
    lane = lax.broadcasted_iota(jnp.int32, (CBLK, 128), 1)
    r2 = rel * rel
    mw = (lane < 3).astype(jnp.float32)
    mm = ((lane >= 3) & (lane < 5)).astype(jnp.float32)
    nw = jnp.sqrt(jnp.sum(r2 * mw, axis=-1, keepdims=True) + 1e-16)
    nm = jnp.sqrt(jnp.sum(r2 * mm, axis=-1, keepdims=True) + 1e-16)
    h = _MM(rel, w16[...]) + nw * wnw[...] + nm * wnm[...] + b1[...]
    h = jax.nn.relu(h)
    h = jax.nn.relu(_MM(h, w2[...]) + b2[...])
    h = _MM(h, w3[...]) + b3[...]
    o[0, 0] = _ln(h, sc[...], of[...])


def _edge_enc(P3, *w):
    nc = P3.shape[1]
    return pl.pallas_call(
        _edge_enc_body,
        grid=(3, 2, nc // CBLK),
        in_specs=[
            pl.BlockSpec((1, CBLK, 128), lambda j, i, cb: ((j + 2 - i) % 3, cb, 0)),
            pl.BlockSpec((1, CBLK, 128), lambda j, i, cb: (j, cb, 0)),
            _wspec(128, 128), _wspec(1, 128), _wspec(1, 128), _wspec(1, 128),
            _wspec(128, 128), _wspec(1, 128), _wspec(128, 128),
            _wspec(1, 128), _wspec(1, 128), _wspec(1, 128),
        ],
        out_specs=pl.BlockSpec((1, 1, CBLK, 128), lambda j, i, cb: (j, i, cb, 0)),
        out_shape=_f32(3, 2, nc, 128),
    )(P3, P3, *w)


def _make_edge_step_body(nreal):
    def body(gr, gs0, gs1, ef, w1a, w1b, w1c, b1, w2, b2, w3, b3,
             sc, of, nef, s):
        tr = _MM(gr[0], w1b[...])
        acc = None
        for i in range(2):
            gs = (gs0 if i == 0 else gs1)[0]
            efi = ef[0, i]
            h = jax.nn.relu(_MM(gs, w1a[...]) + tr + _MM(efi, w1c[...])
                            + b1[...])
            h = jax.nn.relu(_MM(h, w2[...]) + b2[...])
            h = _MM(h, w3[...]) + b3[...]
            y = _ln(h, sc[...], of[...]) + efi
            nef[0, i] = y
            acc = y if acc is None else acc + y
        row = (lax.broadcasted_iota(jnp.int32, (CBLK, 1), 0)
               + pl.program_id(1) * CBLK)
        s[0] = jnp.where(row < nreal, acc, 0.0)
    return body


def _edge_step(body, G3, ef, w1a, w1b, w1c, b1, w2, b2, w3, b3, sc, of):
    nc = G3.shape[1]
    return pl.pallas_call(
        body,
        grid=(3, nc // CBLK),
        in_specs=[
            pl.BlockSpec((1, CBLK, 128), lambda j, cb: (j, cb, 0)),
            pl.BlockSpec((1, CBLK, 128), lambda j, cb: ((j + 2) % 3, cb, 0)),
            pl.BlockSpec((1, CBLK, 128), lambda j, cb: ((j + 1) % 3, cb, 0)),
            pl.BlockSpec((1, 2, CBLK, 128), lambda j, cb: (j, 0, cb, 0)),
            _wspec(128, 128), _wspec(128, 128), _wspec(128, 128),
            _wspec(1, 128), _wspec(128, 128), _wspec(1, 128),
            _wspec(128, 128), _wspec(1, 128), _wspec(1, 128), _wspec(1, 128),
        ],
        out_specs=[
            pl.BlockSpec((1, 2, CBLK, 128), lambda j, cb: (j, 0, cb, 0)),
            pl.BlockSpec((1, CBLK, 128), lambda j, cb: (j, cb, 0)),
        ],
        out_shape=[_f32(3, 2, nc, 128), _f32(3, nc, 128)],
    )(G3, G3, G3, ef, w1a, w1b, w1c, b1, w2, b2, w3, b3, sc, of)


_EDGE_BODY_A = _make_edge_step_body(C)
_EDGE_BODY_B = _make_edge_step_body(C - HP)


def _node_step_body(nf, pa, pb, v1a, v1b, b1, w2, b2, w3, b3, sc, of, o):
    x = nf[...]
    agg = (pa[0] + pa[1]) + (pb[0] + pb[1])
    h = jax.nn.relu(_MM(x, v1a[...]) + _MM(agg, v1b[...]) + b1[...])
    h = jax.nn.relu(_MM(h, w2[...]) + b2[...])
    h = _MM(h, w3[...]) + b3[...]
    o[...] = _ln(h, sc[...], of[...]) + x


def _node_step(nf, pa, pb, *w):
    return pl.pallas_call(
        _node_step_body,
        grid=(NP // NBLK,),
        in_specs=[
            pl.BlockSpec((NBLK, 128), lambda nb: (nb, 0)),
            pl.BlockSpec((2, NBLK, 128), lambda nb: (0, nb, 0)),
            pl.BlockSpec((2, NBLK, 128), lambda nb: (0, nb, 0)),
            _wspec(128, 128), _wspec(128, 128), _wspec(1, 128),
            _wspec(128, 128), _wspec(1, 128), _wspec(128, 128),
            _wspec(1, 128), _wspec(1, 128), _wspec(1, 128),
        ],
        out_specs=pl.BlockSpec((NBLK, 128), lambda nb: (nb, 0)),
        out_shape=_f32(NP, 128),
    )(nf, pa, pb, *w)


def _decode_body(nf, wp, pwp, w1, b1, w2, b2, w3, b3, o):
    h = jax.nn.relu(_MM(nf[...], w1[...]) + b1[...])
    h = jax.nn.relu(_MM(h, w2[...]) + b2[...])
    acc = _MM(h, w3[...]) + b3[...]
    o[...] = 2.0 * wp[...] - pwp[...] + acc


def _decode(nf, wp8, pwp8, *w):
    return pl.pallas_call(
        _decode_body,
        grid=(NP // NBLK,),
        in_specs=[
            pl.BlockSpec((NBLK, 128), lambda nb: (nb, 0)),
            pl.BlockSpec((NBLK, 8), lambda nb: (nb, 0)),
            pl.BlockSpec((NBLK, 8), lambda nb: (nb, 0)),
            _wspec(128, 128), _wspec(1, 128), _wspec(128, 128),
            _wspec(1, 128), _wspec(128, 8), _wspec(1, 8),
        ],
        out_specs=pl.BlockSpec((NBLK, 8), lambda nb: (nb, 0)),
        out_shape=_f32(NP, 8),
    )(nf, wp8, pwp8, *w)


def kernel(world_pos, prev_world_pos, mesh_pos, node_type, cells, is_training,
           params):
    f32 = jnp.float32
    wp8 = jnp.zeros((NP, 8), f32).at[:N, :3].set(world_pos)
    pwp8 = jnp.zeros((NP, 8), f32).at[:N, :3].set(prev_world_pos)
    nt = jnp.zeros((NP, 1), jnp.int32).at[:N].set(node_type)
    pos128 = (jnp.zeros((NP, 128), f32)
              .at[:N, :3].set(world_pos)
              .at[:N, 3:5].set(mesh_pos))
    cells_p = jnp.zeros((CP, 3), jnp.int32).at[:C].set(cells)
    idxA = cells_p[:HP].T.reshape(NW, NCH, RCH)
    idxB = cells_p[HP:].T.reshape(NW, NCH, RCH)

    nmean = params["node_norm"]["mean"]
    nstd = params["node_norm"]["std"]
    (W1, b1), (W2, b2), (W3, b3) = params["node_encoder"]["layers"]
    W1e = W1 / nstd[:, None]
    b1e = b1 - (nmean / nstd) @ W1
    Wv = jnp.zeros((8, 128), f32).at[:3].set(W1e[:3])
    Wo = jnp.zeros((16, 128), f32).at[:9].set(W1e[3:12])
    enc_w = (Wv, Wo, b1e[None], W2, b2[None], W3, b3[None],
             params["node_encoder"]["ln_scale"][None],
             params["node_encoder"]["ln_offset"][None])

    emean = params["edge_norm"]["mean"]
    estd = params["edge_norm"]["std"]
    (Ew1, eb1), (Ew2, eb2), (Ew3, eb3) = params["edge_encoder"]["layers"]
    Ew1e = Ew1 / estd[:, None]
    eb1e = eb1 - (emean / estd) @ Ew1
    W16 = (jnp.zeros((128, 128), f32)
           .at[0:3].set(Ew1e[0:3])
           .at[3:5].set(Ew1e[4:6]))
    wnw = Ew1e[3][None]
    wnm = Ew1e[6][None]
    eenc_w = (W16, wnw, wnm, eb1e[None], Ew2, eb2[None], Ew3, eb3[None],
              params["edge_encoder"]["ln_scale"][None],
              params["edge_encoder"]["ln_offset"][None])

    ostd = params["out_norm"]["std"]
    omean = params["out_norm"]["mean"]
    (D1, d1), (D2, d2), (D3, d3) = params["decoder"]["layers"]
    D3e = jnp.zeros((128, 8), f32).at[:, :3].set(D3 * ostd[None, :])
    d3e = jnp.zeros((8,), f32).at[:3].set(d3 * ostd + omean)
    dec_w = (D1, d1[None], D2, d2[None], D3e, d3e[None])

    nf = _encode(wp8, pwp8, nt, *enc_w)
    PA = _sc_gather(pos128, idxA, 128).reshape(3, HP, 128)
    PB = _sc_gather(pos128, idxB, 128).reshape(3, HP, 128)
    efA = _edge_enc(PA, *eenc_w)
    efB = _edge_enc(PB, *eenc_w)

    widx0 = jnp.zeros((NW, 1, 8), jnp.int32)
    idxA_t, idxB_t = idxA, idxB
    for blk in params["blocks"]:
        (ew1, eb1_), (ew2, eb2_), (ew3, eb3_) = blk["edge"]["layers"]
        (nw1, nb1_), (nw2, nb2_), (nw3, nb3_) = blk["node"]["layers"]
        ew = (ew1[:128], ew1[128:256], ew1[256:], eb1_[None],
              ew2, eb2_[None], ew3, eb3_[None],
              blk["edge"]["ln_scale"][None], blk["edge"]["ln_offset"][None])
        GA = _sc_gather(nf, idxA_t, 128).reshape(3, HP, 128)
        GB = _sc_gather(nf, idxB_t, 128).reshape(3, HP, 128)
        nefA, sA = _edge_step(_EDGE_BODY_A, GA, efA, *ew)
        pA = _sc_scatter(sA.reshape(BH, 128), idxA)
        nefB, sB = _edge_step(_EDGE_BODY_B, GB, efB, *ew)
        pB = _sc_scatter(sB.reshape(BH, 128), idxB)
        widx, _, _ = lax.optimization_barrier(
            (widx0, pA[0, :1, :8], pB[0, :1, :8]))
        warm = _sc_gather(pos128, widx, 128)
        (idxA_t, idxB_t), _ = lax.optimization_barrier(
            ((idxA, idxB), warm[:1, :8]))
        nf = _node_step(
            nf, pA, pB,
            nw1[:128], nw1[128:], nb1_[None],
            nw2, nb2_[None], nw3, nb3_[None],
            blk["node"]["ln_scale"][None], blk["node"]["ln_offset"][None])
        efA, efB = nefA, nefB

    out8 = _decode(nf, wp8, pwp8, *dec_w)
    return out8[:N, :3]


# speedup vs baseline: 1.0006x; 1.0006x over previous
import functools

import jax
import jax.numpy as jnp
from jax import lax
from jax.experimental import pallas as pl
from jax.experimental.pallas import tpu as pltpu
from jax.experimental.pallas import tpu_sc as plsc

N = 10000
C = 20000
NP = 10240
CP = 20480
HP = CP // 2
BH = 3 * HP
NBLK = 1024
CBLK = 2048
NW = 32
RCH = 120
NCH = BH // NW // RCH
PW = NCH * RCH
SLAB = NP // 16
NBUF = 7
NBUF_S = 2


def _MM(x, w):
    bf = jnp.bfloat16
    xh = x.astype(bf)
    xl = (x - xh.astype(jnp.float32)).astype(bf)
    wh = w.astype(bf)
    wl = (w - wh.astype(jnp.float32)).astype(bf)
    d = functools.partial(jnp.dot, preferred_element_type=jnp.float32)
    return d(xh, wh) + (d(xh, wl) + d(xl, wh))


def _ln(h, scale, off):
    mu = jnp.mean(h, axis=-1, keepdims=True)
    d = h - mu
    var = jnp.mean(d * d, axis=-1, keepdims=True)
    return d * lax.rsqrt(var + 1e-5) * scale + off


def _sc_gather(table, idx3d, D):
    _, nch, rch = idx3d.shape
    pw = nch * rch
    mesh = plsc.VectorSubcoreMesh(core_axis_name="c", subcore_axis_name="s")

    def body(table_hbm, idx_hbm, out_hbm, idx_v, rows_v, gsem, osem):
        wid = lax.axis_index("s") * 2 + lax.axis_index("c")
        pltpu.sync_copy(idx_hbm.at[wid], idx_v)
        gd = [None] * NBUF
        od = [None] * NBUF
        lag = NBUF - 1
        for k in range(nch + lag):
            if k < nch:
                b = k % NBUF
                if od[b] is not None:
                    od[b].wait()
                gd[b] = pltpu.async_copy(table_hbm.at[idx_v.at[k]],
                                         rows_v.at[b], gsem.at[b])
            if lag <= k < nch + lag:
                kk = k - lag
                if kk < 0 or kk >= nch:
                    continue
                bb = kk % NBUF
                gd[bb].wait()
                od[bb] = pltpu.async_copy(
                    rows_v.at[bb],
                    out_hbm.at[pl.ds(wid * pw + kk * rch, rch)],
                    osem.at[bb])
        for b in range(NBUF):
            if od[b] is not None:
                od[b].wait()

    f = pl.kernel(
        body,
        out_type=jax.ShapeDtypeStruct((NW * pw, D), jnp.float32),
        mesh=mesh,
        scratch_types=[
            pltpu.VMEM((nch, rch), jnp.int32),
            pltpu.VMEM((NBUF, rch, D), jnp.float32),
            pltpu.SemaphoreType.DMA((NBUF,)),
            pltpu.SemaphoreType.DMA((NBUF,)),
        ],
    )
    return f(table, idx3d)


def _sc_scatter(vals, idx3d):
    mesh = plsc.VectorSubcoreMesh(core_axis_name="c", subcore_axis_name="s")

    def body(vals_hbm, idx_hbm, out_hbm, idx_v, vals_v, acc,
             isem, vsem, asem):
        cid = lax.axis_index("c")
        sid = lax.axis_index("s")
        wid = sid * 2 + cid

        idx_d = pltpu.async_copy(idx_hbm.at[wid], idx_v, isem)

        def zrow(i, carry):
            z = jnp.zeros((16,), jnp.float32)
            for jj in range(8):
                vals_v[0, i, pl.ds(jj * 16, 16)] = z
            return carry

        lax.fori_loop(0, RCH, zrow, 0)
        off = 0
        while off < SLAB:
            sz = min(RCH, SLAB - off)
            pltpu.sync_copy(vals_v.at[0, pl.ds(0, sz)],
                            acc.at[pl.ds(sid * SLAB + off, sz)])
            off += sz
        idx_d.wait()
        plsc.subcore_barrier()

        vd = [None] * NBUF_S
        ad = [None] * NBUF_S
        for k in range(NCH):
            b = k % NBUF_S
            if ad[b] is not None:
                ad[b].wait()
            vd[b] = pltpu.async_copy(
                vals_hbm.at[pl.ds(wid * PW + k * RCH, RCH)],
                vals_v.at[b], vsem.at[b])
            if k >= 1:
                bb = (k - 1) % NBUF_S
                vd[bb].wait()
                ad[bb] = pltpu.async_copy(vals_v.at[bb],
                                          acc.at[idx_v.at[k - 1]],
                                          asem.at[bb], add=True)
        bb = (NCH - 1) % NBUF_S
        vd[bb].wait()
        ad[bb] = pltpu.async_copy(vals_v.at[bb], acc.at[idx_v.at[NCH - 1]],
                                  asem.at[bb], add=True)
        for b in range(NBUF_S):
            if ad[b] is not None:
                ad[b].wait()
        plsc.subcore_barrier()
        pltpu.sync_copy(acc.at[pl.ds(sid * SLAB, SLAB)],
                        out_hbm.at[cid, pl.ds(sid * SLAB, SLAB)])

    f = pl.kernel(
        body,
        out_type=jax.ShapeDtypeStruct((2, NP, 128), jnp.float32),
        mesh=mesh,
        scratch_types=[
            pltpu.VMEM((NCH, RCH), jnp.int32),
            pltpu.VMEM((NBUF_S, RCH, 128), jnp.float32),
            pltpu.VMEM_SHARED((NP, 128), jnp.float32),
            pltpu.SemaphoreType.DMA,
            pltpu.SemaphoreType.DMA((NBUF_S,)),
            pltpu.SemaphoreType.DMA((NBUF_S,)),
        ],
    )
    return f(vals, idx3d)


def _f32(*shape):
    return jax.ShapeDtypeStruct(shape, jnp.float32)


def _wspec(*shp):
    nd = len(shp)
    return pl.BlockSpec(shp, lambda *_: (0,) * nd)


def _encode_body(wp, pwp, nt, wv, wo, b1, w2, b2, w3, b3, sc, of, o):
    vel = wp[...] - pwp[...]
    lane = lax.broadcasted_iota(jnp.int32, (NBLK, 16), 1)
    oh = (lane == nt[...]).astype(jnp.float32)
    h = _MM(vel, wv[...]) + _MM(oh, wo[...]) + b1[...]
    h = jax.nn.relu(h)
    h = jax.nn.relu(_MM(h, w2[...]) + b2[...])
    h = _MM(h, w3[...]) + b3[...]
    o[...] = _ln(h, sc[...], of[...])


def _encode(wp8, pwp8, nt, *w):
    return pl.pallas_call(
        _encode_body,
        grid=(NP // NBLK,),
        in_specs=[
            pl.BlockSpec((NBLK, 8), lambda nb: (nb, 0)),
            pl.BlockSpec((NBLK, 8), lambda nb: (nb, 0)),
            pl.BlockSpec((NBLK, 1), lambda nb: (nb, 0)),
            _wspec(8, 128), _wspec(16, 128), _wspec(1, 128),
            _wspec(128, 128), _wspec(1, 128), _wspec(128, 128),
            _wspec(1, 128), _wspec(1, 128), _wspec(1, 128),
        ],
        out_specs=pl.BlockSpec((NBLK, 128), lambda nb: (nb, 0)),
        out_shape=_f32(NP, 128),
    )(wp8, pwp8, nt, *w)


def _edge_enc_body(ps, pr, w16, wnw, wnm, b1, w2, b2, w3, b3, sc, of, o):
    rel = ps[0] - pr[0]
    lane = lax.broadcasted_iota(jnp.int32, (CBLK, 128), 1)
    r2 = rel * rel
    mw = (lane < 3).astype(jnp.float32)
    mm = ((lane >= 3) & (lane < 5)).astype(jnp.float32)
    nw = jnp.sqrt(jnp.sum(r2 * mw, axis=-1, keepdims=True) + 1e-16)
    nm = jnp.sqrt(jnp.sum(r2 * mm, axis=-1, keepdims=True) + 1e-16)
    h = _MM(rel, w16[...]) + nw * wnw[...] + nm * wnm[...] + b1[...]
    h = jax.nn.relu(h)
    h = jax.nn.relu(_MM(h, w2[...]) + b2[...])
    h = _MM(h, w3[...]) + b3[...]
    o[0, 0] = _ln(h, sc[...], of[...])


def _edge_enc(P3, *w):
    nc = P3.shape[1]
    return pl.pallas_call(
        _edge_enc_body,
        grid=(3, 2, nc // CBLK),
        in_specs=[
            pl.BlockSpec((1, CBLK, 128), lambda j, i, cb: ((j + 2 - i) % 3, cb, 0)),
            pl.BlockSpec((1, CBLK, 128), lambda j, i, cb: (j, cb, 0)),
            _wspec(128, 128), _wspec(1, 128), _wspec(1, 128), _wspec(1, 128),
            _wspec(128, 128), _wspec(1, 128), _wspec(128, 128),
            _wspec(1, 128), _wspec(1, 128), _wspec(1, 128),
        ],
        out_specs=pl.BlockSpec((1, 1, CBLK, 128), lambda j, i, cb: (j, i, cb, 0)),
        out_shape=_f32(3, 2, nc, 128),
    )(P3, P3, *w)


def _make_edge_step_body(nreal):
    def body(gr, gs0, gs1, ef, w1a, w1b, w1c, b1, w2, b2, w3, b3,
             sc, of, nef, s):
        tr = _MM(gr[0], w1b[...])
        acc = None
        for i in range(2):
            gs = (gs0 if i == 0 else gs1)[0]
            efi = ef[0, i]
            h = jax.nn.relu(_MM(gs, w1a[...]) + tr + _MM(efi, w1c[...])
                            + b1[...])
            h = jax.nn.relu(_MM(h, w2[...]) + b2[...])
            h = _MM(h, w3[...]) + b3[...]
            y = _ln(h, sc[...], of[...]) + efi
            nef[0, i] = y
            acc = y if acc is None else acc + y
        row = (lax.broadcasted_iota(jnp.int32, (CBLK, 1), 0)
               + pl.program_id(1) * CBLK)
        s[0] = jnp.where(row < nreal, acc, 0.0)
    return body


def _edge_step(body, G3, ef, w1a, w1b, w1c, b1, w2, b2, w3, b3, sc, of):
    nc = G3.shape[1]
    return pl.pallas_call(
        body,
        grid=(3, nc // CBLK),
        in_specs=[
            pl.BlockSpec((1, CBLK, 128), lambda j, cb: (j, cb, 0)),
            pl.BlockSpec((1, CBLK, 128), lambda j, cb: ((j + 2) % 3, cb, 0)),
            pl.BlockSpec((1, CBLK, 128), lambda j, cb: ((j + 1) % 3, cb, 0)),
            pl.BlockSpec((1, 2, CBLK, 128), lambda j, cb: (j, 0, cb, 0)),
            _wspec(128, 128), _wspec(128, 128), _wspec(128, 128),
            _wspec(1, 128), _wspec(128, 128), _wspec(1, 128),
            _wspec(128, 128), _wspec(1, 128), _wspec(1, 128), _wspec(1, 128),
        ],
        out_specs=[
            pl.BlockSpec((1, 2, CBLK, 128), lambda j, cb: (j, 0, cb, 0)),
            pl.BlockSpec((1, CBLK, 128), lambda j, cb: (j, cb, 0)),
        ],
        out_shape=[_f32(3, 2, nc, 128), _f32(3, nc, 128)],
    )(G3, G3, G3, ef, w1a, w1b, w1c, b1, w2, b2, w3, b3, sc, of)


_EDGE_BODY_A = _make_edge_step_body(C)
_EDGE_BODY_B = _make_edge_step_body(C - HP)


def _node_step_body(nf, pa, pb, v1a, v1b, b1, w2, b2, w3, b3, sc, of, o):
    x = nf[...]
    agg = (pa[0] + pa[1]) + (pb[0] + pb[1])
    h = jax.nn.relu(_MM(x, v1a[...]) + _MM(agg, v1b[...]) + b1[...])
    h = jax.nn.relu(_MM(h, w2[...]) + b2[...])
    h = _MM(h, w3[...]) + b3[...]
    o[...] = _ln(h, sc[...], of[...]) + x


def _node_step(nf, pa, pb, *w):
    return pl.pallas_call(
        _node_step_body,
        grid=(NP // NBLK,),
        in_specs=[
            pl.BlockSpec((NBLK, 128), lambda nb: (nb, 0)),
            pl.BlockSpec((2, NBLK, 128), lambda nb: (0, nb, 0)),
            pl.BlockSpec((2, NBLK, 128), lambda nb: (0, nb, 0)),
            _wspec(128, 128), _wspec(128, 128), _wspec(1, 128),
            _wspec(128, 128), _wspec(1, 128), _wspec(128, 128),
            _wspec(1, 128), _wspec(1, 128), _wspec(1, 128),
        ],
        out_specs=pl.BlockSpec((NBLK, 128), lambda nb: (nb, 0)),
        out_shape=_f32(NP, 128),
    )(nf, pa, pb, *w)


def _decode_body(nf, wp, pwp, w1, b1, w2, b2, w3, b3, o):
    h = jax.nn.relu(_MM(nf[...], w1[...]) + b1[...])
    h = jax.nn.relu(_MM(h, w2[...]) + b2[...])
    acc = _MM(h, w3[...]) + b3[...]
    o[...] = 2.0 * wp[...] - pwp[...] + acc


def _decode(nf, wp8, pwp8, *w):
    return pl.pallas_call(
        _decode_body,
        grid=(NP // NBLK,),
        in_specs=[
            pl.BlockSpec((NBLK, 128), lambda nb: (nb, 0)),
            pl.BlockSpec((NBLK, 8), lambda nb: (nb, 0)),
            pl.BlockSpec((NBLK, 8), lambda nb: (nb, 0)),
            _wspec(128, 128), _wspec(1, 128), _wspec(128, 128),
            _wspec(1, 128), _wspec(128, 8), _wspec(1, 8),
        ],
        out_specs=pl.BlockSpec((NBLK, 8), lambda nb: (nb, 0)),
        out_shape=_f32(NP, 8),
    )(nf, wp8, pwp8, *w)


def kernel(world_pos, prev_world_pos, mesh_pos, node_type, cells, is_training,
           params):
    f32 = jnp.float32
    wp8 = jnp.zeros((NP, 8), f32).at[:N, :3].set(world_pos)
    pwp8 = jnp.zeros((NP, 8), f32).at[:N, :3].set(prev_world_pos)
    nt = jnp.zeros((NP, 1), jnp.int32).at[:N].set(node_type)
    pos128 = (jnp.zeros((NP, 128), f32)
              .at[:N, :3].set(world_pos)
              .at[:N, 3:5].set(mesh_pos))
    cells_p = jnp.zeros((CP, 3), jnp.int32).at[:C].set(cells)
    idxA = cells_p[:HP].T.reshape(NW, NCH, RCH)
    idxB = cells_p[HP:].T.reshape(NW, NCH, RCH)

    nmean = params["node_norm"]["mean"]
    nstd = params["node_norm"]["std"]
    (W1, b1), (W2, b2), (W3, b3) = params["node_encoder"]["layers"]
    W1e = W1 / nstd[:, None]
    b1e = b1 - (nmean / nstd) @ W1
    Wv = jnp.zeros((8, 128), f32).at[:3].set(W1e[:3])
    Wo = jnp.zeros((16, 128), f32).at[:9].set(W1e[3:12])
    enc_w = (Wv, Wo, b1e[None], W2, b2[None], W3, b3[None],
             params["node_encoder"]["ln_scale"][None],
             params["node_encoder"]["ln_offset"][None])

    emean = params["edge_norm"]["mean"]
    estd = params["edge_norm"]["std"]
    (Ew1, eb1), (Ew2, eb2), (Ew3, eb3) = params["edge_encoder"]["layers"]
    Ew1e = Ew1 / estd[:, None]
    eb1e = eb1 - (emean / estd) @ Ew1
    W16 = (jnp.zeros((128, 128), f32)
           .at[0:3].set(Ew1e[0:3])
           .at[3:5].set(Ew1e[4:6]))
    wnw = Ew1e[3][None]
    wnm = Ew1e[6][None]
    eenc_w = (W16, wnw, wnm, eb1e[None], Ew2, eb2[None], Ew3, eb3[None],
              params["edge_encoder"]["ln_scale"][None],
              params["edge_encoder"]["ln_offset"][None])

    ostd = params["out_norm"]["std"]
    omean = params["out_norm"]["mean"]
    (D1, d1), (D2, d2), (D3, d3) = params["decoder"]["layers"]
    D3e = jnp.zeros((128, 8), f32).at[:, :3].set(D3 * ostd[None, :])
    d3e = jnp.zeros((8,), f32).at[:3].set(d3 * ostd + omean)
    dec_w = (D1, d1[None], D2, d2[None], D3e, d3e[None])

    nf = _encode(wp8, pwp8, nt, *enc_w)
    PA = _sc_gather(pos128, idxA, 128).reshape(3, HP, 128)
    PB = _sc_gather(pos128, idxB, 128).reshape(3, HP, 128)
    efA = _edge_enc(PA, *eenc_w)
    efB = _edge_enc(PB, *eenc_w)

    for blk in params["blocks"]:
        (ew1, eb1_), (ew2, eb2_), (ew3, eb3_) = blk["edge"]["layers"]
        (nw1, nb1_), (nw2, nb2_), (nw3, nb3_) = blk["node"]["layers"]
        ew = (ew1[:128], ew1[128:256], ew1[256:], eb1_[None],
              ew2, eb2_[None], ew3, eb3_[None],
              blk["edge"]["ln_scale"][None], blk["edge"]["ln_offset"][None])
        GA = _sc_gather(nf, idxA, 128).reshape(3, HP, 128)
        GB = _sc_gather(nf, idxB, 128).reshape(3, HP, 128)
        nefA, sA = _edge_step(_EDGE_BODY_A, GA, efA, *ew)
        pA = _sc_scatter(sA.reshape(BH, 128), idxA)
        nefB, sB = _edge_step(_EDGE_BODY_B, GB, efB, *ew)
        pB = _sc_scatter(sB.reshape(BH, 128), idxB)
        nf = _node_step(
            nf, pA, pB,
            nw1[:128], nw1[128:], nb1_[None],
            nw2, nb2_[None], nw3, nb3_[None],
            blk["node"]["ln_scale"][None], blk["node"]["ln_offset"][None])
        efA, efB = nefA, nefB

    out8 = _decode(nf, wp8, pwp8, *dec_w)
    return out8[:N, :3]
